# baseline (speedup 1.0000x reference)
import functools

import jax
import jax.numpy as jnp
from jax import lax
from jax.experimental import pallas as pl
from jax.experimental.pallas import tpu as pltpu
from jax.experimental.pallas import tpu_sc as plsc

NFIELDS = 26
NFACTOR = 16
ROW_OFFSET = 100000
NC = 2
NS = 16
LANES = 16
NW = NC * NS

CHUNK_ROWS = 16
IPC = CHUNK_ROWS * NFIELDS
DMA_SPLIT = 4
DMA_LEN = IPC // DMA_SPLIT


def _fm_body(nchunks, in_hbm, patt_hbm, emb_hbm, fc_hbm, out_hbm,
             in_v, idx_v, patt_v, rows_v, fc_v, o16_v, sem):
    wid = lax.axis_index("s") * NC + lax.axis_index("c")
    base_row = wid * (nchunks * CHUNK_ROWS)
    pltpu.sync_copy(patt_hbm, patt_v)

    iota = lax.iota(jnp.int32, LANES)
    row_iota = iota * NFIELDS

    def chunk_body(c, carry):
        row0 = base_row + c * CHUNK_ROWS
        flat0 = row0 * NFIELDS
        pltpu.sync_copy(in_hbm.at[pl.ds(flat0, IPC)], in_v)
        for g in range(IPC // LANES):
            sl = pl.ds(g * LANES, LANES)
            idx_v[sl] = in_v[sl] + patt_v[sl]
        copies = []
        for i in range(DMA_SPLIT):
            sl = pl.ds(i * DMA_LEN, DMA_LEN)
            copies.append(pltpu.async_copy(
                emb_hbm.at[idx_v.at[sl]], rows_v.at[sl], sem))
            copies.append(pltpu.async_copy(
                fc_hbm.at[idx_v.at[sl]], fc_v.at[sl], sem))
        for cp in copies:
            cp.wait()

        lin = jnp.zeros((LANES,), jnp.float32)
        xvs = []
        for j in range(NFIELDS):
            idxs_j = row_iota + j
            xvs.append(plsc.load_gather(in_v, [idxs_j]).astype(jnp.float32))
            lin = lin + plsc.load_gather(fc_v, [idxs_j])

        q = jnp.zeros((LANES,), jnp.float32)
        for f in range(NFACTOR):
            col = jnp.full((LANES,), f, jnp.int32)
            s = jnp.zeros((LANES,), jnp.float32)
            ss = jnp.zeros((LANES,), jnp.float32)
            for j in range(NFIELDS):
                v = plsc.load_gather(rows_v, [row_iota + j, col])
                t = v * xvs[j]
                s = s + t
                ss = ss + t * t
            q = q + (s * s - ss)

        o16_v[...] = lin + 0.5 * q
        pltpu.sync_copy(o16_v, out_hbm.at[pl.ds(row0, CHUNK_ROWS)])
        return carry

    lax.fori_loop(0, nchunks, chunk_body, 0)


def kernel(input, emb_table, fc_table, global_bias):
    batch = input.shape[0]
    nchunks = batch // (NW * CHUNK_ROWS)
    assert batch == nchunks * NW * CHUNK_ROWS

    in_flat = input.reshape(-1)
    fc_flat = fc_table.reshape(-1)
    pattern = jnp.tile(jnp.arange(NFIELDS, dtype=jnp.int32) * ROW_OFFSET,
                       CHUNK_ROWS)

    mesh = plsc.VectorSubcoreMesh(core_axis_name="c", subcore_axis_name="s",
                                  num_cores=NC, num_subcores=NS)
    fm = pl.kernel(
        functools.partial(_fm_body, nchunks),
        out_type=jax.ShapeDtypeStruct((batch,), jnp.float32),
        mesh=mesh,
        compiler_params=pltpu.CompilerParams(needs_layout_passes=False,
                                             use_tc_tiling_on_sc=False),
        scratch_types=[
            pltpu.VMEM((IPC,), jnp.int32),
            pltpu.VMEM((IPC,), jnp.int32),
            pltpu.VMEM((IPC,), jnp.int32),
            pltpu.VMEM((IPC, NFACTOR), jnp.float32),
            pltpu.VMEM((IPC,), jnp.float32),
            pltpu.VMEM((LANES,), jnp.float32),
            pltpu.SemaphoreType.DMA,
        ],
    )
    out = fm(in_flat, pattern, emb_table, fc_flat)
    return out + global_bias[0]


# speedup vs baseline: 1.0301x; 1.0301x over previous
import functools

import jax
import jax.numpy as jnp
from jax import lax
from jax.experimental import pallas as pl
from jax.experimental.pallas import tpu as pltpu
from jax.experimental.pallas import tpu_sc as plsc

NFIELDS = 26
NFACTOR = 16
ROW_OFFSET = 100000
NC = 2
NS = 16
LANES = 16
NW = NC * NS

CHUNK_ROWS = 16
IPC = CHUNK_ROWS * NFIELDS
DMA_SPLIT = 4
DMA_LEN = IPC // DMA_SPLIT


def _fm_body(nchunks, in_hbm, patt_hbm, emb_hbm, fc_hbm, out_hbm,
             in_v0, idx_v0, rows_v0, fc_v0,
             in_v1, idx_v1, rows_v1, fc_v1,
             patt_v, o16_v, sem0, sem1):
    wid = lax.axis_index("s") * NC + lax.axis_index("c")
    base_row = wid * (nchunks * CHUNK_ROWS)
    pltpu.sync_copy(patt_hbm, patt_v)

    iota = lax.iota(jnp.int32, LANES)
    row_iota = iota * NFIELDS
    bufs = ((in_v0, idx_v0, rows_v0, fc_v0, sem0),
            (in_v1, idx_v1, rows_v1, fc_v1, sem1))

    def stage(c, buf):
        in_v, idx_v, rows_v, fc_v, sem = buf
        flat0 = (base_row + c * CHUNK_ROWS) * NFIELDS
        pltpu.sync_copy(in_hbm.at[pl.ds(flat0, IPC)], in_v)
        for g in range(IPC // LANES):
            sl = pl.ds(g * LANES, LANES)
            idx_v[sl] = in_v[sl] + patt_v[sl]
        for i in range(DMA_SPLIT):
            sl = pl.ds(i * DMA_LEN, DMA_LEN)
            pltpu.async_copy(emb_hbm.at[idx_v.at[sl]], rows_v.at[sl], sem)
            pltpu.async_copy(fc_hbm.at[idx_v.at[sl]], fc_v.at[sl], sem)

    def drain(buf):
        in_v, idx_v, rows_v, fc_v, sem = buf
        for i in range(DMA_SPLIT):
            sl = pl.ds(i * DMA_LEN, DMA_LEN)
            pltpu.make_async_copy(emb_hbm.at[idx_v.at[sl]],
                                  rows_v.at[sl], sem).wait()
            pltpu.make_async_copy(fc_hbm.at[idx_v.at[sl]],
                                  fc_v.at[sl], sem).wait()

    def compute(c, buf):
        in_v, idx_v, rows_v, fc_v, sem = buf
        lin = jnp.zeros((LANES,), jnp.float32)
        s = [jnp.zeros((LANES,), jnp.float32) for _ in range(NFACTOR)]
        ss = [jnp.zeros((LANES,), jnp.float32) for _ in range(NFACTOR)]
        for j in range(NFIELDS):
            idxs_j = row_iota + j
            xv = plsc.load_gather(in_v, [idxs_j]).astype(jnp.float32)
            lin = lin + plsc.load_gather(fc_v, [idxs_j])
            for f in range(NFACTOR):
                v = plsc.load_gather(
                    rows_v, [idxs_j, jnp.full((LANES,), f, jnp.int32)])
                t = v * xv
                s[f] = s[f] + t
                ss[f] = ss[f] + t * t
        q = jnp.zeros((LANES,), jnp.float32)
        for f in range(NFACTOR):
            q = q + (s[f] * s[f] - ss[f])
        o16_v[...] = lin + 0.5 * q
        pltpu.sync_copy(
            o16_v, out_hbm.at[pl.ds(base_row + c * CHUNK_ROWS, CHUNK_ROWS)])

    stage(0, bufs[0])

    def pair_body(i, carry):
        c0 = i * 2
        stage(c0 + 1, bufs[1])
        drain(bufs[0])
        compute(c0, bufs[0])

        @pl.when(c0 + 2 < nchunks)
        def _():
            stage(c0 + 2, bufs[0])

        drain(bufs[1])
        compute(c0 + 1, bufs[1])
        return carry

    lax.fori_loop(0, nchunks // 2, pair_body, 0)


def kernel(input, emb_table, fc_table, global_bias):
    batch = input.shape[0]
    nchunks = batch // (NW * CHUNK_ROWS)
    assert batch == nchunks * NW * CHUNK_ROWS and nchunks % 2 == 0

    in_flat = input.reshape(-1)
    fc_flat = fc_table.reshape(-1)
    pattern = jnp.tile(jnp.arange(NFIELDS, dtype=jnp.int32) * ROW_OFFSET,
                       CHUNK_ROWS)

    mesh = plsc.VectorSubcoreMesh(core_axis_name="c", subcore_axis_name="s",
                                  num_cores=NC, num_subcores=NS)
    fm = pl.kernel(
        functools.partial(_fm_body, nchunks),
        out_type=jax.ShapeDtypeStruct((batch,), jnp.float32),
        mesh=mesh,
        compiler_params=pltpu.CompilerParams(needs_layout_passes=False,
                                             use_tc_tiling_on_sc=False),
        scratch_types=[
            pltpu.VMEM((IPC,), jnp.int32),
            pltpu.VMEM((IPC,), jnp.int32),
            pltpu.VMEM((IPC, NFACTOR), jnp.float32),
            pltpu.VMEM((IPC,), jnp.float32),
            pltpu.VMEM((IPC,), jnp.int32),
            pltpu.VMEM((IPC,), jnp.int32),
            pltpu.VMEM((IPC, NFACTOR), jnp.float32),
            pltpu.VMEM((IPC,), jnp.float32),
            pltpu.VMEM((IPC,), jnp.int32),
            pltpu.VMEM((LANES,), jnp.float32),
            pltpu.SemaphoreType.DMA,
            pltpu.SemaphoreType.DMA,
        ],
    )
    out = fm(in_flat, pattern, emb_table, fc_flat)
    return out + global_bias[0]
